# baseline (speedup 1.0000x reference)
import jax
import jax.numpy as jnp
from jax import lax
from jax.experimental import pallas as pl
from jax.experimental.pallas import tpu as pltpu
from jax.experimental.pallas import tpu_sc as plsc

N_FIELDS = 26
VOCAB = 100000
EMB = 32
BATCH = 16384

NC = 2
NS = 16
NW = NC * NS
B_PER_W = BATCH // NW
CHUNK = 128
SUB = B_PER_W // CHUNK
NCHUNKS = N_FIELDS * SUB
NBUF = 4
LANES = 16


def _tec_body(*refs):
  tbls = refs[:N_FIELDS]
  idx_hbm, out_hbm, idx_v, gbuf, acc, ld_sem, g_sems = refs[N_FIELDS:]
  wid = lax.axis_index("s") * NC + lax.axis_index("c")

  pltpu.async_copy(idx_hbm.at[wid], idx_v, ld_sem).wait()

  def _fire(ch, b):
    pltpu.async_copy(tbls[ch // SUB].at[idx_v.at[ch]], gbuf.at[b],
                     g_sems.at[b])

  for b in range(NBUF):
    _fire(b, b)

  zero = jnp.zeros((LANES,), jnp.float32)

  def _zero(r, c):
    for u in range(8):
      acc[r * 8 + u, pl.ds(0, LANES)] = zero
      acc[r * 8 + u, pl.ds(LANES, LANES)] = zero
    return c

  lax.fori_loop(0, B_PER_W // 8, _zero, 0, unroll=False)

  for ch in range(NCHUNKS):
    b = ch % NBUF
    pltpu.make_async_copy(tbls[ch // SUB].at[idx_v.at[ch]], gbuf.at[b],
                          g_sems.at[b]).wait()

    base = (ch % SUB) * CHUNK

    def _accum(r, c, b=b, base=base):
      row = base + r * 4
      for u in range(4):
        g0 = gbuf[b, r * 4 + u, pl.ds(0, LANES)]
        g1 = gbuf[b, r * 4 + u, pl.ds(LANES, LANES)]
        plsc.addupdate(acc.at[row + u, pl.ds(0, LANES)], g0)
        plsc.addupdate(acc.at[row + u, pl.ds(LANES, LANES)], g1)
      return c

    lax.fori_loop(0, CHUNK // 4, _accum, 0, unroll=False)

    if ch + NBUF < NCHUNKS:
      _fire(ch + NBUF, b)

  pltpu.async_copy(acc, out_hbm.at[pl.ds(wid * B_PER_W, B_PER_W)],
                   ld_sem).wait()


@jax.jit
def kernel(x, tables):
  idx = x.reshape(NW, SUB, CHUNK, N_FIELDS).transpose(0, 3, 1, 2)
  idx = idx.reshape(NW, NCHUNKS, CHUNK)

  mesh = plsc.VectorSubcoreMesh(core_axis_name="c", subcore_axis_name="s")
  f = pl.kernel(
      _tec_body,
      out_type=jax.ShapeDtypeStruct((BATCH, EMB), jnp.float32),
      mesh=mesh,
      compiler_params=pltpu.CompilerParams(use_tc_tiling_on_sc=False),
      scratch_types=[
          pltpu.VMEM((NCHUNKS, CHUNK), jnp.int32),
          pltpu.VMEM((NBUF, CHUNK, EMB), jnp.float32),
          pltpu.VMEM((B_PER_W, EMB), jnp.float32),
          pltpu.SemaphoreType.DMA,
          pltpu.SemaphoreType.DMA((NBUF,)),
      ],
  )
  return f(*[tables[i] for i in range(N_FIELDS)], idx)


# speedup vs baseline: 1.6069x; 1.6069x over previous
import jax
import jax.numpy as jnp
from jax import lax
from jax.experimental import pallas as pl
from jax.experimental.pallas import tpu as pltpu
from jax.experimental.pallas import tpu_sc as plsc

N_FIELDS = 26
VOCAB = 100000
EMB = 32
BATCH = 16384

NC = 2
NS = 16
NW = NC * NS
B_PER_W = BATCH // NW
CHUNK = 128
SUB = B_PER_W // CHUNK
NCHUNKS = N_FIELDS * SUB
NBUF = 4
LANES = 16


def _tec_body(idx_hbm, tbl_hbm, out_hbm, idx_v, gbuf, acc, ld_sem, g_sems):
  wid = lax.axis_index("s") * NC + lax.axis_index("c")

  pltpu.async_copy(idx_hbm.at[wid], idx_v, ld_sem).wait()

  for b in range(NBUF):
    pltpu.async_copy(tbl_hbm.at[idx_v.at[b]], gbuf.at[b], g_sems.at[b])

  zero = jnp.zeros((LANES,), jnp.float32)

  def _zero(r, c):
    for u in range(8):
      acc[r * 8 + u, pl.ds(0, LANES)] = zero
      acc[r * 8 + u, pl.ds(LANES, LANES)] = zero
    return c

  lax.fori_loop(0, B_PER_W // 8, _zero, 0, unroll=False)

  def _step(ch, b):
    pltpu.make_async_copy(tbl_hbm.at[idx_v.at[ch]], gbuf.at[b],
                          g_sems.at[b]).wait()

    base = (ch % SUB) * CHUNK

    def _accum(r, c):
      row = base + r * 4
      for u in range(4):
        g0 = gbuf[b, r * 4 + u, pl.ds(0, LANES)]
        g1 = gbuf[b, r * 4 + u, pl.ds(LANES, LANES)]
        plsc.addupdate(acc.at[row + u, pl.ds(0, LANES)], g0)
        plsc.addupdate(acc.at[row + u, pl.ds(LANES, LANES)], g1)
      return c

    lax.fori_loop(0, CHUNK // 4, _accum, 0, unroll=False)

    nxt = ch + NBUF

    @pl.when(nxt < NCHUNKS)
    def _():
      pltpu.async_copy(tbl_hbm.at[idx_v.at[nxt]], gbuf.at[b], g_sems.at[b])

  def _ring(j, c):
    for b in range(NBUF):
      _step(j * NBUF + b, b)
    return c

  lax.fori_loop(0, NCHUNKS // NBUF, _ring, 0, unroll=False)

  pltpu.async_copy(acc, out_hbm.at[pl.ds(wid * B_PER_W, B_PER_W)],
                   ld_sem).wait()


@jax.jit
def kernel(x, tables):
  tbl_flat = tables.reshape(N_FIELDS * VOCAB, EMB)

  flat_idx = x.astype(jnp.int32) + (jnp.arange(N_FIELDS, dtype=jnp.int32)
                                    * VOCAB)[None, :]
  idx = flat_idx.reshape(NW, SUB, CHUNK, N_FIELDS).transpose(0, 3, 1, 2)
  idx = idx.reshape(NW, NCHUNKS, CHUNK)

  mesh = plsc.VectorSubcoreMesh(core_axis_name="c", subcore_axis_name="s")
  f = pl.kernel(
      _tec_body,
      out_type=jax.ShapeDtypeStruct((BATCH, EMB), jnp.float32),
      mesh=mesh,
      compiler_params=pltpu.CompilerParams(use_tc_tiling_on_sc=False),
      scratch_types=[
          pltpu.VMEM((NCHUNKS, CHUNK), jnp.int32),
          pltpu.VMEM((NBUF, CHUNK, EMB), jnp.float32),
          pltpu.VMEM((B_PER_W, EMB), jnp.float32),
          pltpu.SemaphoreType.DMA,
          pltpu.SemaphoreType.DMA((NBUF,)),
      ],
  )
  return f(idx, tbl_flat)
